# baseline (speedup 1.0000x reference)
import functools

import jax
import jax.numpy as jnp
from jax import lax
from jax.experimental import pallas as pl
from jax.experimental.pallas import tpu as pltpu
from jax.experimental.pallas import tpu_sc as plsc

DIM = 32
NW = 32
CHUNK = 128
BLOCKS = CHUNK // 16


def _make(b_total, interpret=False):
  per_w = b_total // NW
  n_chunks = per_w // CHUNK
  mesh = plsc.VectorSubcoreMesh(
      core_axis_name="c", subcore_axis_name="s", num_cores=2, num_subcores=16)

  @functools.partial(
      pl.kernel,
      out_type=jax.ShapeDtypeStruct((b_total, DIM), jnp.float32),
      mesh=mesh,
      interpret=interpret,
      compiler_params=pltpu.CompilerParams(
          needs_layout_passes=False, use_tc_tiling_on_sc=False),
      scratch_types=[
          pltpu.VMEM((2, CHUNK), jnp.int32),
          pltpu.VMEM((2, CHUNK, DIM), jnp.float32),
          pltpu.VMEM((2, CHUNK, 2), jnp.float32),
          pltpu.VMEM((2, CHUNK, DIM), jnp.float32),
          pltpu.SemaphoreType.DMA,
          pltpu.SemaphoreType.DMA,
          pltpu.SemaphoreType.DMA,
          pltpu.SemaphoreType.DMA,
          pltpu.SemaphoreType.DMA,
          pltpu.SemaphoreType.DMA,
          pltpu.SemaphoreType.DMA,
          pltpu.SemaphoreType.DMA,
      ],
  )
  def k(q_hbm, sz_hbm, idx_hbm, out_hbm, ci_v, rows_v, sz_v, o_v,
        si0, si1, sq0, sq1, ss0, ss1, so0, so1):
    wid = lax.axis_index("s") * 2 + lax.axis_index("c")
    base = wid * per_w
    si = (si0, si1)
    sq = (sq0, sq1)
    ss = (ss0, ss1)
    so = (so0, so1)
    ci = (ci_v.at[0], ci_v.at[1])
    rows = (rows_v.at[0], rows_v.at[1])
    szs = (sz_v.at[0], sz_v.at[1])
    o = (o_v.at[0], o_v.at[1])

    lanes = lax.iota(jnp.int32, 16)
    zeros = jnp.zeros((16,), jnp.int32)
    ones = jnp.ones((16,), jnp.int32)

    def idx_slice(k_):
      return idx_hbm.at[pl.ds(base + k_ * CHUNK, CHUNK)]

    def out_slice(k_):
      return out_hbm.at[pl.ds(base + k_ * CHUNK, CHUNK), :]

    def issue_gathers(p):
      pltpu.async_copy(q_hbm.at[ci[p]], rows[p], sq[p])
      pltpu.async_copy(sz_hbm.at[ci[p]], szs[p], ss[p])

    def wait_gathers(p):
      pltpu.make_async_copy(q_hbm.at[ci[p]], rows[p], sq[p]).wait()
      pltpu.make_async_copy(sz_hbm.at[ci[p]], szs[p], ss[p]).wait()

    def compute_chunk(rows_ref, sz_ref, o_ref):
      @plsc.parallel_loop(0, BLOCKS, unroll=1)
      def _block(bi):
        row = lanes + bi * 16
        sv = plsc.load_gather(sz_ref, [row, zeros])
        zv = plsc.load_gather(sz_ref, [row, ones])
        for e in range(DIM):
          col = jnp.full((16,), e, jnp.int32)
          w = plsc.load_gather(rows_ref, [row, col])
          plsc.store_scatter(o_ref, [row, col], (w - zv) * sv)

    pltpu.sync_copy(idx_slice(0), ci[0])
    issue_gathers(0)
    pltpu.async_copy(idx_slice(1), ci[1], si[1])

    def half_iter(jj, carry):
      for p in (0, 1):
        kk = jj * 2 + p
        q_ = p ^ 1

        @pl.when(kk + 1 < n_chunks)
        def _():
          pltpu.make_async_copy(idx_slice(kk + 1), ci[q_], si[q_]).wait()
          issue_gathers(q_)

        wait_gathers(p)

        @pl.when(kk + 2 < n_chunks)
        def _():
          pltpu.async_copy(idx_slice(kk + 2), ci[p], si[p])

        @pl.when(kk >= 2)
        def _():
          pltpu.make_async_copy(o[p], out_slice(kk), so[p]).wait()

        compute_chunk(rows[p], szs[p], o[p])
        pltpu.async_copy(o[p], out_slice(kk), so[p])
      return carry

    lax.fori_loop(0, n_chunks // 2, half_iter, 0)
    pltpu.make_async_copy(o[0], out_slice(0), so[0]).wait()
    pltpu.make_async_copy(o[1], out_slice(1), so[1]).wait()

  return k


def kernel(input, q_weight, scale, zero_point):
  b_total = input.shape[0] * input.shape[1]
  idx = jnp.asarray(input, jnp.int32).reshape(b_total)
  qf = q_weight.astype(jnp.float32)
  sz = jnp.stack([scale.astype(jnp.float32),
                  zero_point.astype(jnp.float32)], axis=1)
  out = _make(b_total)(qf, sz, idx)
  return out.reshape(input.shape[0], input.shape[1], DIM)


# speedup vs baseline: 2.4439x; 2.4439x over previous
import functools

import jax
import jax.numpy as jnp
from jax import lax
from jax.experimental import pallas as pl
from jax.experimental.pallas import tpu as pltpu
from jax.experimental.pallas import tpu_sc as plsc

DIM = 32
NW = 32
CHUNK = 128
BLOCKS = CHUNK // 16


def _make(b_total, interpret=False):
  per_w = b_total // NW
  n_chunks = per_w // CHUNK
  mesh = plsc.VectorSubcoreMesh(
      core_axis_name="c", subcore_axis_name="s", num_cores=2, num_subcores=16)

  @functools.partial(
      pl.kernel,
      out_type=jax.ShapeDtypeStruct((b_total, DIM), jnp.float32),
      mesh=mesh,
      interpret=interpret,
      compiler_params=pltpu.CompilerParams(
          needs_layout_passes=False, use_tc_tiling_on_sc=False),
      scratch_types=[
          pltpu.VMEM((2, CHUNK), jnp.int32),
          pltpu.VMEM((2, CHUNK, DIM), jnp.float32),
          pltpu.VMEM((2, CHUNK), jnp.float32),
          pltpu.VMEM((2, CHUNK), jnp.float32),
          pltpu.VMEM((2, CHUNK, DIM), jnp.float32),
          pltpu.SemaphoreType.DMA,
          pltpu.SemaphoreType.DMA,
          pltpu.SemaphoreType.DMA,
          pltpu.SemaphoreType.DMA,
          pltpu.SemaphoreType.DMA,
          pltpu.SemaphoreType.DMA,
          pltpu.SemaphoreType.DMA,
          pltpu.SemaphoreType.DMA,
          pltpu.SemaphoreType.DMA,
          pltpu.SemaphoreType.DMA,
      ],
  )
  def k(q_hbm, s_hbm, z_hbm, idx_hbm, out_hbm, ci_v, rows_v, sc_v, zp_v, o_v,
        si0, si1, sq0, sq1, ss0, ss1, sz0, sz1, so0, so1):
    wid = lax.axis_index("s") * 2 + lax.axis_index("c")
    base = wid * per_w
    si = (si0, si1)
    sq = (sq0, sq1)
    ss = (ss0, ss1)
    sz = (sz0, sz1)
    so = (so0, so1)
    ci = (ci_v.at[0], ci_v.at[1])
    rows = (rows_v.at[0], rows_v.at[1])
    scs = (sc_v.at[0], sc_v.at[1])
    zps = (zp_v.at[0], zp_v.at[1])
    o = (o_v.at[0], o_v.at[1])

    lanes = lax.iota(jnp.int32, 16)

    def idx_slice(k_):
      return idx_hbm.at[pl.ds(base + k_ * CHUNK, CHUNK)]

    def out_slice(k_):
      return out_hbm.at[pl.ds(base + k_ * CHUNK, CHUNK), :]

    def issue_gathers(p):
      pltpu.async_copy(q_hbm.at[ci[p]], rows[p], sq[p])
      pltpu.async_copy(s_hbm.at[ci[p]], scs[p], ss[p])
      pltpu.async_copy(z_hbm.at[ci[p]], zps[p], sz[p])

    def wait_gathers(p):
      pltpu.make_async_copy(q_hbm.at[ci[p]], rows[p], sq[p]).wait()
      pltpu.make_async_copy(s_hbm.at[ci[p]], scs[p], ss[p]).wait()
      pltpu.make_async_copy(z_hbm.at[ci[p]], zps[p], sz[p]).wait()

    def compute_chunk(rows_ref, s_ref, z_ref, o_ref):
      @plsc.parallel_loop(0, BLOCKS, unroll=1)
      def _block(bi):
        row = lanes + bi * 16
        sv = plsc.load_gather(s_ref, [row])
        zv = plsc.load_gather(z_ref, [row])
        for e in range(DIM):
          col = jnp.full((16,), e, jnp.int32)
          w = plsc.load_gather(rows_ref, [row, col])
          plsc.store_scatter(o_ref, [row, col], (w - zv) * sv)

    pltpu.sync_copy(idx_slice(0), ci[0])
    issue_gathers(0)
    pltpu.async_copy(idx_slice(1), ci[1], si[1])

    def half_iter(jj, carry):
      for p in (0, 1):
        kk = jj * 2 + p
        q_ = p ^ 1

        @pl.when(kk + 1 < n_chunks)
        def _():
          pltpu.make_async_copy(idx_slice(kk + 1), ci[q_], si[q_]).wait()
          issue_gathers(q_)

        wait_gathers(p)

        @pl.when(kk + 2 < n_chunks)
        def _():
          pltpu.async_copy(idx_slice(kk + 2), ci[p], si[p])

        @pl.when(kk >= 2)
        def _():
          pltpu.make_async_copy(o[p], out_slice(kk), so[p]).wait()

        compute_chunk(rows[p], scs[p], zps[p], o[p])
        pltpu.async_copy(o[p], out_slice(kk), so[p])
      return carry

    lax.fori_loop(0, n_chunks // 2, half_iter, 0)
    pltpu.make_async_copy(o[0], out_slice(0), so[0]).wait()
    pltpu.make_async_copy(o[1], out_slice(1), so[1]).wait()

  return k


def kernel(input, q_weight, scale, zero_point):
  b_total = input.shape[0] * input.shape[1]
  idx = jnp.asarray(input, jnp.int32).reshape(b_total)
  qf = q_weight.astype(jnp.float32)
  out = _make(b_total)(qf, scale.astype(jnp.float32),
                       zero_point.astype(jnp.float32), idx)
  return out.reshape(input.shape[0], input.shape[1], DIM)
